# Initial kernel scaffold; baseline (speedup 1.0000x reference)
import functools

import jax
import jax.numpy as jnp
from jax import lax
from jax.experimental import pallas as pl
from jax.experimental.pallas import tpu as pltpu
from jax.experimental.pallas import tpu_sc as plsc

_NC, _NS, _L = 2, 16, 16
_NW = _NC * _NS
_B, _F = 16384, 26
_R = _B // _NW
_K = _R * _F
_CH = 128
_NCHUNK = _K // _CH


def _body(idx_hbm, w_hbm, out_hbm, idx_v, vals_v, out_v, sem):
    wid = lax.axis_index("s") * _NC + lax.axis_index("c")
    base = pl.multiple_of(wid * _K, _K)
    pltpu.sync_copy(idx_hbm.at[pl.ds(base, _K)], idx_v)

    def fire(j, c):
        off = pl.multiple_of(j * _CH, _CH)
        pltpu.async_copy(
            w_hbm.at[idx_v.at[pl.ds(off, _CH)]],
            vals_v.at[pl.ds(off, _CH)],
            sem,
        )
        return c

    lax.fori_loop(0, _NCHUNK, fire, 0)
    pltpu.make_async_copy(w_hbm.at[pl.ds(0, _K)], vals_v, sem).wait()

    lanes = lax.iota(jnp.int32, _L) * _F

    def red(j, c):
        p0 = lanes + j * (_L * _F)
        acc = plsc.load_gather(vals_v, [p0])
        for f in range(1, _F):
            acc = acc + plsc.load_gather(vals_v, [p0 + f])
        out_v[pl.ds(pl.multiple_of(j * _L, _L), _L)] = acc
        return c

    lax.fori_loop(0, _R // _L, red, 0)
    pltpu.sync_copy(out_v, out_hbm.at[pl.ds(pl.multiple_of(wid * _R, _R), _R)])


_sc_call = pl.kernel(
    _body,
    out_type=jax.ShapeDtypeStruct((_B,), jnp.float32),
    mesh=plsc.VectorSubcoreMesh(
        core_axis_name="c", subcore_axis_name="s",
        num_cores=_NC, num_subcores=_NS,
    ),
    scratch_types=[
        pltpu.VMEM((_K,), jnp.int32),
        pltpu.VMEM((_K,), jnp.float32),
        pltpu.VMEM((_R,), jnp.float32),
        pltpu.SemaphoreType.DMA,
    ],
)


@jax.jit
def kernel(inputs, w):
    idx = inputs.astype(jnp.int32).reshape(-1)
    table = w.reshape(-1)
    return _sc_call(idx, table).reshape(_B, 1)


# speedup vs baseline: 1.3273x; 1.3273x over previous
import functools

import jax
import jax.numpy as jnp
from jax import lax
from jax.experimental import pallas as pl
from jax.experimental.pallas import tpu as pltpu
from jax.experimental.pallas import tpu_sc as plsc

_NC, _NS, _L = 2, 16, 16
_NW = _NC * _NS
_B, _F = 16384, 26
_R = _B // _NW
_K = _R * _F
_CH = 128
_NCHUNK = _K // _CH


def _body(idx_hbm, w_hbm, out_hbm, idx_v, vals_v, out_v, sem):
    wid = lax.axis_index("s") * _NC + lax.axis_index("c")
    base = pl.multiple_of(wid * _K, _K)
    pltpu.sync_copy(idx_hbm.at[pl.ds(base, _K)], idx_v)

    def fire(j, c):
        off = pl.multiple_of(j * _CH, _CH)
        pltpu.async_copy(
            w_hbm.at[idx_v.at[pl.ds(off, _CH)]],
            vals_v.at[pl.ds(off, _CH)],
            sem,
        )
        return c

    lax.fori_loop(0, _NCHUNK, fire, 0)
    pltpu.make_async_copy(w_hbm.at[pl.ds(0, _K)], vals_v, sem).wait()

    lanes = lax.iota(jnp.int32, _L) * _F

    def red(j, c):
        p0 = lanes + j * (_L * _F)
        acc = plsc.load_gather(vals_v, [p0])
        for f in range(1, _F):
            acc = acc + plsc.load_gather(vals_v, [p0 + f])
        out_v[pl.ds(pl.multiple_of(j * _L, _L), _L)] = acc
        return c

    lax.fori_loop(0, _R // _L, red, 0)
    pltpu.sync_copy(out_v, out_hbm.at[pl.ds(pl.multiple_of(wid * _R, _R), _R)])


_sc_call = pl.kernel(
    _body,
    out_type=jax.ShapeDtypeStruct((_B,), jnp.float32),
    mesh=plsc.VectorSubcoreMesh(
        core_axis_name="c", subcore_axis_name="s",
        num_cores=_NC, num_subcores=_NS,
    ),
    scratch_types=[
        pltpu.VMEM((_K,), jnp.int32),
        pltpu.VMEM((_K,), jnp.float32),
        pltpu.VMEM((_R,), jnp.float32),
        pltpu.SemaphoreType.DMA,
    ],
    compiler_params=pltpu.CompilerParams(needs_layout_passes=False),
)


@jax.jit
def kernel(inputs, w):
    idx = inputs.astype(jnp.int32).reshape(-1)
    table = w.reshape(-1)
    return _sc_call(idx, table).reshape(_B, 1)
